# Initial kernel scaffold; baseline (speedup 1.0000x reference)
import jax
import jax.numpy as jnp
from jax.experimental import pallas as pl


def kernel(edge_index, x, W1, b1, W2, b2, W3, b3):
    raise NotImplementedError("write your pallas kernel here")


# speedup vs baseline: 2.8942x; 2.8942x over previous
import functools

import jax
import jax.numpy as jnp
from jax import lax
from jax.experimental import pallas as pl
from jax.experimental.pallas import tpu as pltpu
from jax.experimental.pallas import tpu_sc as plsc

N = 10000
NP = 10240
E = 320000
CH = 128
NCH = 80
EPT = NCH * CH
NT = 32
PE = NT * EPT
RPT = NP // 16
BR = 1024

_mesh = plsc.VectorSubcoreMesh(core_axis_name="c", subcore_axis_name="s")


def _make_agg(F):
    @functools.partial(
        pl.kernel,
        mesh=_mesh,
        out_type=jax.ShapeDtypeStruct((2 * NP, F), jnp.float32),
        scratch_types=[
            pltpu.VMEM((NCH, CH), jnp.int32),
            pltpu.VMEM((NCH, CH), jnp.int32),
            pltpu.VMEM((CH, F), jnp.float32),
            pltpu.VMEM_SHARED((NP, F), jnp.float32),
            pltpu.SemaphoreType.DMA,
        ],
    )
    def agg(src_hbm, dst_hbm, h_hbm, zz_hbm, out_hbm, idx_s, idx_d, rows, acc, sem):
        cid = lax.axis_index("c")
        sid = lax.axis_index("s")
        tid = cid * 16 + sid
        pltpu.sync_copy(zz_hbm, rows)
        for k in range(RPT // CH):
            pltpu.sync_copy(rows, acc.at[pl.ds(sid * RPT + k * CH, CH)])
        pltpu.sync_copy(src_hbm.at[tid], idx_s)
        pltpu.sync_copy(dst_hbm.at[tid], idx_d)
        plsc.subcore_barrier()

        def body(j, carry):
            pltpu.async_copy(h_hbm.at[idx_s.at[j]], rows, sem).wait()
            pltpu.sync_copy(rows, acc.at[idx_d.at[j]], add=True)
            return carry

        lax.fori_loop(0, NCH, body, 0)
        plsc.subcore_barrier()
        for k in range(RPT // CH):
            r0 = sid * RPT + k * CH
            pltpu.sync_copy(acc.at[pl.ds(r0, CH)], rows)
            pltpu.sync_copy(rows, out_hbm.at[pl.ds(cid * NP + r0, CH)])

    return agg


_agg = _make_agg(128)


@functools.partial(
    pl.kernel,
    mesh=_mesh,
    out_type=(
        jax.ShapeDtypeStruct((2 * NP,), jnp.float32),
        jax.ShapeDtypeStruct((2 * NP,), jnp.float32),
    ),
    scratch_types=[
        pltpu.VMEM((NCH, CH), jnp.int32),
        pltpu.VMEM((NCH, CH), jnp.int32),
        pltpu.VMEM((CH,), jnp.float32),
        pltpu.VMEM((RPT,), jnp.float32),
        pltpu.VMEM_SHARED((NP,), jnp.float32),
        pltpu.VMEM_SHARED((NP,), jnp.float32),
    ],
)
def _deg(src_hbm, dst_hbm, ones_hbm, zz_hbm, out_o, out_i,
         idx_s, idx_d, ones_v, tmp, acc_o, acc_i):
    cid = lax.axis_index("c")
    sid = lax.axis_index("s")
    tid = cid * 16 + sid
    pltpu.sync_copy(ones_hbm, ones_v)
    pltpu.sync_copy(zz_hbm, tmp)
    pltpu.sync_copy(tmp, acc_o.at[pl.ds(sid * RPT, RPT)])
    pltpu.sync_copy(tmp, acc_i.at[pl.ds(sid * RPT, RPT)])
    pltpu.sync_copy(src_hbm.at[tid], idx_s)
    pltpu.sync_copy(dst_hbm.at[tid], idx_d)
    plsc.subcore_barrier()

    def body(j, carry):
        pltpu.sync_copy(ones_v, acc_o.at[idx_s.at[j]], add=True)
        pltpu.sync_copy(ones_v, acc_i.at[idx_d.at[j]], add=True)
        return carry

    lax.fori_loop(0, NCH, body, 0)
    plsc.subcore_barrier()
    pltpu.sync_copy(acc_o.at[pl.ds(sid * RPT, RPT)], tmp)
    pltpu.sync_copy(tmp, out_o.at[pl.ds(cid * NP + sid * RPT, RPT)])
    pltpu.sync_copy(acc_i.at[pl.ds(sid * RPT, RPT)], tmp)
    pltpu.sync_copy(tmp, out_i.at[pl.ds(cid * NP + sid * RPT, RPT)])


def _elu(v):
    return jnp.where(v > 0, v, jnp.exp(jnp.minimum(v, 0.0)) - 1.0)


def _norm(d_ref):
    d = d_ref[0] + d_ref[1]
    return lax.rsqrt(jnp.maximum(d, 1.0))


def _tc1_body(x_ref, w_ref, do_ref, o_ref):
    ns = _norm(do_ref)
    o_ref[...] = jnp.dot(x_ref[...] * ns, w_ref[...],
                         preferred_element_type=jnp.float32)


def _tc2_body(a_ref, b_ref, di_ref, do_ref, o_ref):
    nd = _norm(di_ref)
    ns = _norm(do_ref)
    a = a_ref[0] + a_ref[1]
    o_ref[...] = _elu(a * nd + b_ref[...]) * ns


def _tc3_body(a_ref, w2_ref, b2_ref, w3_ref, di_ref, do_ref, o_ref):
    nd = _norm(di_ref)
    ns = _norm(do_ref)
    a = a_ref[0] + a_ref[1]
    h = _elu(jnp.dot(a * nd, w2_ref[...], preferred_element_type=jnp.float32)
             + b2_ref[...])
    o_ref[...] = jnp.dot(h * ns, w3_ref[...], preferred_element_type=jnp.float32)


def _tc4_body(a_ref, b_ref, di_ref, o_ref):
    nd = _norm(di_ref)
    a = a_ref[0] + a_ref[1]
    v = a * nd + b_ref[...]
    o_ref[...] = 1.0 / (1.0 + jnp.exp(-v))


def _deg_spec():
    return pl.BlockSpec((2, BR, 1), lambda i: (0, i, 0))


def _resh2(a):
    return a.reshape(2, NP, a.shape[-1])


def kernel(edge_index, x, W1, b1, W2, b2, W3, b3):
    src = edge_index[0].astype(jnp.int32)
    dst = edge_index[1].astype(jnp.int32)
    pad_e = PE - E
    srcp = jnp.concatenate([src, jnp.full((pad_e,), N, jnp.int32)]).reshape(NT, NCH, CH)
    dstp = jnp.concatenate([dst, jnp.full((pad_e,), N, jnp.int32)]).reshape(NT, NCH, CH)
    xp = jnp.pad(x, ((0, NP - N), (0, 0)))
    W1p = jnp.pad(W1, ((0, 0), (0, 28)))
    b1p = jnp.pad(b1, (0, 28)).reshape(1, 128)
    W2p = jnp.pad(W2, ((0, 28), (0, 0)))
    b2r = b2.reshape(1, 200)
    b3r = b3.reshape(1, 128)
    ones_c = jnp.ones((CH,), jnp.float32)
    zz_r = jnp.zeros((RPT,), jnp.float32)
    zz128 = jnp.zeros((CH, 128), jnp.float32)

    deg_o, deg_i = _deg(srcp, dstp, ones_c, zz_r)
    deg_o = deg_o.reshape(2, NP, 1)
    deg_i = deg_i.reshape(2, NP, 1)

    t1 = pl.pallas_call(
        _tc1_body,
        grid=(NP // BR,),
        in_specs=[
            pl.BlockSpec((BR, 128), lambda i: (i, 0)),
            pl.BlockSpec((128, 128), lambda i: (0, 0)),
            _deg_spec(),
        ],
        out_specs=pl.BlockSpec((BR, 128), lambda i: (i, 0)),
        out_shape=jax.ShapeDtypeStruct((NP, 128), jnp.float32),
    )(xp, W1p, deg_o)

    a1 = _resh2(_agg(srcp, dstp, t1, zz128))

    u2 = pl.pallas_call(
        _tc2_body,
        grid=(NP // BR,),
        in_specs=[
            pl.BlockSpec((2, BR, 128), lambda i: (0, i, 0)),
            pl.BlockSpec((1, 128), lambda i: (0, 0)),
            _deg_spec(),
            _deg_spec(),
        ],
        out_specs=pl.BlockSpec((BR, 128), lambda i: (i, 0)),
        out_shape=jax.ShapeDtypeStruct((NP, 128), jnp.float32),
    )(a1, b1p, deg_i, deg_o)

    a2 = _resh2(_agg(srcp, dstp, u2, zz128))

    t3 = pl.pallas_call(
        _tc3_body,
        grid=(NP // BR,),
        in_specs=[
            pl.BlockSpec((2, BR, 128), lambda i: (0, i, 0)),
            pl.BlockSpec((128, 200), lambda i: (0, 0)),
            pl.BlockSpec((1, 200), lambda i: (0, 0)),
            pl.BlockSpec((200, 128), lambda i: (0, 0)),
            _deg_spec(),
            _deg_spec(),
        ],
        out_specs=pl.BlockSpec((BR, 128), lambda i: (i, 0)),
        out_shape=jax.ShapeDtypeStruct((NP, 128), jnp.float32),
    )(a2, W2p, b2r, W3, deg_i, deg_o)

    a3 = _resh2(_agg(srcp, dstp, t3, zz128))

    out = pl.pallas_call(
        _tc4_body,
        grid=(NP // BR,),
        in_specs=[
            pl.BlockSpec((2, BR, 128), lambda i: (0, i, 0)),
            pl.BlockSpec((1, 128), lambda i: (0, 0)),
            _deg_spec(),
        ],
        out_specs=pl.BlockSpec((BR, 128), lambda i: (i, 0)),
        out_shape=jax.ShapeDtypeStruct((NP, 128), jnp.float32),
    )(a3, b3r, deg_i)

    return out[:N]


# speedup vs baseline: 3.2851x; 1.1351x over previous
import functools

import jax
import jax.numpy as jnp
from jax import lax
from jax.experimental import pallas as pl
from jax.experimental.pallas import tpu as pltpu
from jax.experimental.pallas import tpu_sc as plsc

N = 10000
NP = 10240
E = 320000
CH = 128
NCH = 80
EPT = NCH * CH
NT = 32
PE = NT * EPT
TCH = PE // CH
K0 = 40
K1 = 120
CH0 = 16 * K0
RPT = NP // 16
BR = 1024

_mesh = plsc.VectorSubcoreMesh(core_axis_name="c", subcore_axis_name="s")


def _make_agg(F):
    @functools.partial(
        pl.kernel,
        mesh=_mesh,
        out_type=jax.ShapeDtypeStruct((2 * NP, F), jnp.float32),
        scratch_types=[
            pltpu.VMEM((K1, CH), jnp.int32),
            pltpu.VMEM((K1, CH), jnp.int32),
            pltpu.VMEM((CH, F), jnp.float32),
            pltpu.VMEM_SHARED((NP, F), jnp.float32),
            pltpu.SemaphoreType.DMA,
        ],
    )
    def agg(src_hbm, dst_hbm, h_hbm, zz_hbm, out_hbm,
            idx_s, idx_d, rows0, acc, sem0):
        cid = lax.axis_index("c")
        sid = lax.axis_index("s")
        base = jnp.where(cid == 0, sid * K0, CH0 + sid * K1)
        ncht = jnp.where(cid == 0, K0, K1)
        pltpu.sync_copy(zz_hbm, rows0)
        for k in range(RPT // CH):
            pltpu.sync_copy(rows0, acc.at[pl.ds(sid * RPT + k * CH, CH)])
        pltpu.sync_copy(src_hbm.at[pl.ds(base, K1)], idx_s)
        pltpu.sync_copy(dst_hbm.at[pl.ds(base, K1)], idx_d)
        plsc.subcore_barrier()

        def body(j, carry):
            pltpu.async_copy(h_hbm.at[idx_s.at[j]], rows0, sem0).wait()
            pltpu.sync_copy(rows0, acc.at[idx_d.at[j]], add=True)
            return carry

        lax.fori_loop(0, ncht, body, 0)
        plsc.subcore_barrier()
        for k in range(RPT // CH):
            r0 = sid * RPT + k * CH
            pltpu.sync_copy(acc.at[pl.ds(r0, CH)], rows0)
            pltpu.sync_copy(rows0, out_hbm.at[pl.ds(cid * NP + r0, CH)])

    return agg


_agg = _make_agg(128)


@functools.partial(
    pl.kernel,
    mesh=_mesh,
    out_type=(
        jax.ShapeDtypeStruct((2 * NP,), jnp.float32),
        jax.ShapeDtypeStruct((2 * NP,), jnp.float32),
    ),
    scratch_types=[
        pltpu.VMEM((NCH, CH), jnp.int32),
        pltpu.VMEM((NCH, CH), jnp.int32),
        pltpu.VMEM((CH,), jnp.float32),
        pltpu.VMEM((RPT,), jnp.float32),
        pltpu.VMEM_SHARED((NP,), jnp.float32),
        pltpu.VMEM_SHARED((NP,), jnp.float32),
    ],
)
def _deg(src_hbm, dst_hbm, ones_hbm, zz_hbm, out_o, out_i,
         idx_s, idx_d, ones_v, tmp, acc_o, acc_i):
    cid = lax.axis_index("c")
    sid = lax.axis_index("s")
    tid = cid * 16 + sid
    pltpu.sync_copy(ones_hbm, ones_v)
    pltpu.sync_copy(zz_hbm, tmp)
    pltpu.sync_copy(tmp, acc_o.at[pl.ds(sid * RPT, RPT)])
    pltpu.sync_copy(tmp, acc_i.at[pl.ds(sid * RPT, RPT)])
    pltpu.sync_copy(src_hbm.at[pl.ds(tid * NCH, NCH)], idx_s)
    pltpu.sync_copy(dst_hbm.at[pl.ds(tid * NCH, NCH)], idx_d)
    plsc.subcore_barrier()

    def body(j, carry):
        pltpu.sync_copy(ones_v, acc_o.at[idx_s.at[j]], add=True)
        pltpu.sync_copy(ones_v, acc_i.at[idx_d.at[j]], add=True)
        return carry

    lax.fori_loop(0, NCH, body, 0)
    plsc.subcore_barrier()
    pltpu.sync_copy(acc_o.at[pl.ds(sid * RPT, RPT)], tmp)
    pltpu.sync_copy(tmp, out_o.at[pl.ds(cid * NP + sid * RPT, RPT)])
    pltpu.sync_copy(acc_i.at[pl.ds(sid * RPT, RPT)], tmp)
    pltpu.sync_copy(tmp, out_i.at[pl.ds(cid * NP + sid * RPT, RPT)])


def _elu(v):
    return jnp.where(v > 0, v, jnp.exp(jnp.minimum(v, 0.0)) - 1.0)


def _norm(d_ref):
    d = d_ref[0] + d_ref[1]
    return lax.rsqrt(jnp.maximum(d, 1.0))


def _tc1_body(x_ref, w_ref, do_ref, o_ref):
    ns = _norm(do_ref)
    o_ref[...] = jnp.dot(x_ref[...] * ns, w_ref[...],
                         preferred_element_type=jnp.float32)


def _tc2_body(a_ref, b_ref, di_ref, do_ref, o_ref):
    nd = _norm(di_ref)
    ns = _norm(do_ref)
    a = a_ref[0] + a_ref[1]
    o_ref[...] = _elu(a * nd + b_ref[...]) * ns


def _tc3_body(a_ref, w2_ref, b2_ref, w3_ref, di_ref, do_ref, o_ref):
    nd = _norm(di_ref)
    ns = _norm(do_ref)
    a = a_ref[0] + a_ref[1]
    h = _elu(jnp.dot(a * nd, w2_ref[...], preferred_element_type=jnp.float32)
             + b2_ref[...])
    o_ref[...] = jnp.dot(h * ns, w3_ref[...], preferred_element_type=jnp.float32)


def _tc4_body(a_ref, b_ref, di_ref, o_ref):
    nd = _norm(di_ref)
    a = a_ref[0] + a_ref[1]
    v = a * nd + b_ref[...]
    o_ref[...] = 1.0 / (1.0 + jnp.exp(-v))


def _deg_spec():
    return pl.BlockSpec((2, BR, 1), lambda i: (0, i, 0))


def _resh2(a):
    return a.reshape(2, NP, a.shape[-1])


def kernel(edge_index, x, W1, b1, W2, b2, W3, b3):
    src = edge_index[0].astype(jnp.int32)
    dst = edge_index[1].astype(jnp.int32)
    pad_e = PE - E
    srcp = jnp.concatenate([src, jnp.full((pad_e,), N, jnp.int32)]).reshape(TCH, CH)
    dstp = jnp.concatenate([dst, jnp.full((pad_e,), N, jnp.int32)]).reshape(TCH, CH)
    xp = jnp.pad(x, ((0, NP - N), (0, 0)))
    W1p = jnp.pad(W1, ((0, 0), (0, 28)))
    b1p = jnp.pad(b1, (0, 28)).reshape(1, 128)
    W2p = jnp.pad(W2, ((0, 28), (0, 0)))
    b2r = b2.reshape(1, 200)
    b3r = b3.reshape(1, 128)
    ones_c = jnp.ones((CH,), jnp.float32)
    zz_r = jnp.zeros((RPT,), jnp.float32)
    zz128 = jnp.zeros((CH, 128), jnp.float32)

    deg_o, deg_i = _deg(srcp, dstp, ones_c, zz_r)
    deg_o = deg_o.reshape(2, NP, 1)
    deg_i = deg_i.reshape(2, NP, 1)

    t1 = pl.pallas_call(
        _tc1_body,
        grid=(NP // BR,),
        in_specs=[
            pl.BlockSpec((BR, 128), lambda i: (i, 0)),
            pl.BlockSpec((128, 128), lambda i: (0, 0)),
            _deg_spec(),
        ],
        out_specs=pl.BlockSpec((BR, 128), lambda i: (i, 0)),
        out_shape=jax.ShapeDtypeStruct((NP, 128), jnp.float32),
    )(xp, W1p, deg_o)

    a1 = _resh2(_agg(srcp, dstp, t1, zz128))

    u2 = pl.pallas_call(
        _tc2_body,
        grid=(NP // BR,),
        in_specs=[
            pl.BlockSpec((2, BR, 128), lambda i: (0, i, 0)),
            pl.BlockSpec((1, 128), lambda i: (0, 0)),
            _deg_spec(),
            _deg_spec(),
        ],
        out_specs=pl.BlockSpec((BR, 128), lambda i: (i, 0)),
        out_shape=jax.ShapeDtypeStruct((NP, 128), jnp.float32),
    )(a1, b1p, deg_i, deg_o)

    a2 = _resh2(_agg(srcp, dstp, u2, zz128))

    t3 = pl.pallas_call(
        _tc3_body,
        grid=(NP // BR,),
        in_specs=[
            pl.BlockSpec((2, BR, 128), lambda i: (0, i, 0)),
            pl.BlockSpec((128, 200), lambda i: (0, 0)),
            pl.BlockSpec((1, 200), lambda i: (0, 0)),
            pl.BlockSpec((200, 128), lambda i: (0, 0)),
            _deg_spec(),
            _deg_spec(),
        ],
        out_specs=pl.BlockSpec((BR, 128), lambda i: (i, 0)),
        out_shape=jax.ShapeDtypeStruct((NP, 128), jnp.float32),
    )(a2, W2p, b2r, W3, deg_i, deg_o)

    a3 = _resh2(_agg(srcp, dstp, t3, zz128))

    out = pl.pallas_call(
        _tc4_body,
        grid=(NP // BR,),
        in_specs=[
            pl.BlockSpec((2, BR, 128), lambda i: (0, i, 0)),
            pl.BlockSpec((1, 128), lambda i: (0, 0)),
            _deg_spec(),
        ],
        out_specs=pl.BlockSpec((BR, 128), lambda i: (i, 0)),
        out_shape=jax.ShapeDtypeStruct((NP, 128), jnp.float32),
    )(a3, b3r, deg_i)

    return out[:N]


# speedup vs baseline: 8.1527x; 2.4817x over previous
import functools

import jax
import jax.numpy as jnp
from jax import lax
from jax.experimental import pallas as pl
from jax.experimental.pallas import tpu as pltpu
from jax.experimental.pallas import tpu_sc as plsc

N = 10000
NP = 10240
E = 320000
CH = 128
NCH = 80
EPT = NCH * CH
NT = 32
PE = NT * EPT
TCH = PE // CH
RPT = NP // 16
BR = 1024

_mesh = plsc.VectorSubcoreMesh(core_axis_name="c", subcore_axis_name="s")


def _make_agg(F):
    @functools.partial(
        pl.kernel,
        mesh=_mesh,
        out_type=jax.ShapeDtypeStruct((2 * NP, F), jnp.float32),
        scratch_types=[
            pltpu.VMEM((NCH, CH), jnp.int32),
            pltpu.VMEM((NCH, CH), jnp.int32),
            pltpu.VMEM((CH, F), jnp.float32),
            pltpu.VMEM_SHARED((NP, F), jnp.float32),
            pltpu.SemaphoreType.DMA,
        ],
    )
    def agg(src_hbm, dst_hbm, h_hbm, zz_hbm, out_hbm,
            idx_s, idx_d, rows0, acc, sem0):
        cid = lax.axis_index("c")
        sid = lax.axis_index("s")
        tid = cid * 16 + sid
        pltpu.sync_copy(zz_hbm, rows0)
        for k in range(RPT // CH):
            pltpu.sync_copy(rows0, acc.at[pl.ds(sid * RPT + k * CH, CH)])
        pltpu.sync_copy(src_hbm.at[pl.ds(tid * NCH, NCH)], idx_s)
        pltpu.sync_copy(dst_hbm.at[pl.ds(tid * NCH, NCH)], idx_d)
        plsc.subcore_barrier()

        def body(j, carry):
            pltpu.async_copy(h_hbm.at[idx_s.at[j]], rows0, sem0).wait()
            pltpu.sync_copy(rows0, acc.at[idx_d.at[j]], add=True)
            return carry

        lax.fori_loop(0, NCH, body, 0)
        plsc.subcore_barrier()
        for k in range(RPT // CH):
            r0 = sid * RPT + k * CH
            pltpu.sync_copy(acc.at[pl.ds(r0, CH)], rows0)
            pltpu.sync_copy(rows0, out_hbm.at[pl.ds(cid * NP + r0, CH)])

    return agg


_agg = _make_agg(128)


@functools.partial(
    pl.kernel,
    mesh=_mesh,
    out_type=(
        jax.ShapeDtypeStruct((2 * NP,), jnp.float32),
        jax.ShapeDtypeStruct((2 * NP,), jnp.float32),
    ),
    scratch_types=[
        pltpu.VMEM((NCH, CH), jnp.int32),
        pltpu.VMEM((NCH, CH), jnp.int32),
        pltpu.VMEM((CH,), jnp.float32),
        pltpu.VMEM((RPT,), jnp.float32),
        pltpu.VMEM_SHARED((NP,), jnp.float32),
        pltpu.VMEM_SHARED((NP,), jnp.float32),
    ],
)
def _deg(src_hbm, dst_hbm, ones_hbm, zz_hbm, out_o, out_i,
         idx_s, idx_d, ones_v, tmp, acc_o, acc_i):
    cid = lax.axis_index("c")
    sid = lax.axis_index("s")
    tid = cid * 16 + sid
    pltpu.sync_copy(ones_hbm, ones_v)
    pltpu.sync_copy(zz_hbm, tmp)
    pltpu.sync_copy(tmp, acc_o.at[pl.ds(sid * RPT, RPT)])
    pltpu.sync_copy(tmp, acc_i.at[pl.ds(sid * RPT, RPT)])
    pltpu.sync_copy(src_hbm.at[pl.ds(tid * NCH, NCH)], idx_s)
    pltpu.sync_copy(dst_hbm.at[pl.ds(tid * NCH, NCH)], idx_d)
    plsc.subcore_barrier()

    def body(j, carry):
        pltpu.sync_copy(ones_v, acc_o.at[idx_s.at[j]], add=True)
        pltpu.sync_copy(ones_v, acc_i.at[idx_d.at[j]], add=True)
        return carry

    lax.fori_loop(0, NCH, body, 0)
    plsc.subcore_barrier()
    pltpu.sync_copy(acc_o.at[pl.ds(sid * RPT, RPT)], tmp)
    pltpu.sync_copy(tmp, out_o.at[pl.ds(cid * NP + sid * RPT, RPT)])
    pltpu.sync_copy(acc_i.at[pl.ds(sid * RPT, RPT)], tmp)
    pltpu.sync_copy(tmp, out_i.at[pl.ds(cid * NP + sid * RPT, RPT)])


def _elu(v):
    return jnp.where(v > 0, v, jnp.exp(jnp.minimum(v, 0.0)) - 1.0)


def _norm(d_ref):
    d = d_ref[0] + d_ref[1]
    return lax.rsqrt(jnp.maximum(d, 1.0))


def _tc1_body(x_ref, w_ref, do_ref, o_ref):
    ns = _norm(do_ref)
    o_ref[...] = jnp.dot(x_ref[...] * ns, w_ref[...],
                         preferred_element_type=jnp.float32)


def _tc2_body(a_ref, b_ref, di_ref, do_ref, o_ref):
    nd = _norm(di_ref)
    ns = _norm(do_ref)
    a = a_ref[0] + a_ref[1]
    o_ref[...] = _elu(a * nd + b_ref[...]) * ns


def _tc3_body(a_ref, w2_ref, b2_ref, w3_ref, di_ref, do_ref, o_ref):
    nd = _norm(di_ref)
    ns = _norm(do_ref)
    a = a_ref[0] + a_ref[1]
    h = _elu(jnp.dot(a * nd, w2_ref[...], preferred_element_type=jnp.float32)
             + b2_ref[...])
    o_ref[...] = jnp.dot(h * ns, w3_ref[...], preferred_element_type=jnp.float32)


def _tc4_body(a_ref, b_ref, di_ref, o_ref):
    nd = _norm(di_ref)
    a = a_ref[0] + a_ref[1]
    v = a * nd + b_ref[...]
    o_ref[...] = 1.0 / (1.0 + jnp.exp(-v))


def _deg_spec():
    return pl.BlockSpec((2, BR, 1), lambda i: (0, i, 0))


def _resh2(a):
    return a.reshape(2, NP, a.shape[-1])


def kernel(edge_index, x, W1, b1, W2, b2, W3, b3):
    src = edge_index[0].astype(jnp.int32)
    dst = edge_index[1].astype(jnp.int32)
    pad_e = PE - E
    pad_row = N + jnp.arange(pad_e, dtype=jnp.int32) % (NP - N)
    srcp = jnp.concatenate([src, pad_row]).reshape(TCH, CH)
    dstp = jnp.concatenate([dst, pad_row]).reshape(TCH, CH)
    xp = jnp.pad(x, ((0, NP - N), (0, 0)))
    W1p = jnp.pad(W1, ((0, 0), (0, 28)))
    b1p = jnp.pad(b1, (0, 28)).reshape(1, 128)
    W2p = jnp.pad(W2, ((0, 28), (0, 0)))
    b2r = b2.reshape(1, 200)
    b3r = b3.reshape(1, 128)
    ones_c = jnp.ones((CH,), jnp.float32)
    zz_r = jnp.zeros((RPT,), jnp.float32)
    zz128 = jnp.zeros((CH, 128), jnp.float32)

    deg_o, deg_i = _deg(srcp, dstp, ones_c, zz_r)
    deg_o = deg_o.reshape(2, NP, 1)
    deg_i = deg_i.reshape(2, NP, 1)

    t1 = pl.pallas_call(
        _tc1_body,
        grid=(NP // BR,),
        in_specs=[
            pl.BlockSpec((BR, 128), lambda i: (i, 0)),
            pl.BlockSpec((128, 128), lambda i: (0, 0)),
            _deg_spec(),
        ],
        out_specs=pl.BlockSpec((BR, 128), lambda i: (i, 0)),
        out_shape=jax.ShapeDtypeStruct((NP, 128), jnp.float32),
    )(xp, W1p, deg_o)

    a1 = _resh2(_agg(srcp, dstp, t1, zz128))

    u2 = pl.pallas_call(
        _tc2_body,
        grid=(NP // BR,),
        in_specs=[
            pl.BlockSpec((2, BR, 128), lambda i: (0, i, 0)),
            pl.BlockSpec((1, 128), lambda i: (0, 0)),
            _deg_spec(),
            _deg_spec(),
        ],
        out_specs=pl.BlockSpec((BR, 128), lambda i: (i, 0)),
        out_shape=jax.ShapeDtypeStruct((NP, 128), jnp.float32),
    )(a1, b1p, deg_i, deg_o)

    a2 = _resh2(_agg(srcp, dstp, u2, zz128))

    t3 = pl.pallas_call(
        _tc3_body,
        grid=(NP // BR,),
        in_specs=[
            pl.BlockSpec((2, BR, 128), lambda i: (0, i, 0)),
            pl.BlockSpec((128, 200), lambda i: (0, 0)),
            pl.BlockSpec((1, 200), lambda i: (0, 0)),
            pl.BlockSpec((200, 128), lambda i: (0, 0)),
            _deg_spec(),
            _deg_spec(),
        ],
        out_specs=pl.BlockSpec((BR, 128), lambda i: (i, 0)),
        out_shape=jax.ShapeDtypeStruct((NP, 128), jnp.float32),
    )(a2, W2p, b2r, W3, deg_i, deg_o)

    a3 = _resh2(_agg(srcp, dstp, t3, zz128))

    out = pl.pallas_call(
        _tc4_body,
        grid=(NP // BR,),
        in_specs=[
            pl.BlockSpec((2, BR, 128), lambda i: (0, i, 0)),
            pl.BlockSpec((1, 128), lambda i: (0, 0)),
            _deg_spec(),
        ],
        out_specs=pl.BlockSpec((BR, 128), lambda i: (i, 0)),
        out_shape=jax.ShapeDtypeStruct((NP, 128), jnp.float32),
    )(a3, b3r, deg_i)

    return out[:N]


# speedup vs baseline: 9.1610x; 1.1237x over previous
import functools

import jax
import jax.numpy as jnp
from jax import lax
from jax.experimental import pallas as pl
from jax.experimental.pallas import tpu as pltpu
from jax.experimental.pallas import tpu_sc as plsc

N = 10000
NP = 10240
E = 320000
CH = 128
NCH = 80
EPT = NCH * CH
NT = 32
PE = NT * EPT
TCH = PE // CH
RPT = NP // 16
BR = 1024

_mesh = plsc.VectorSubcoreMesh(core_axis_name="c", subcore_axis_name="s")


def _make_agg(F):
    @functools.partial(
        pl.kernel,
        mesh=_mesh,
        out_type=jax.ShapeDtypeStruct((2 * NP, F), jnp.float32),
        scratch_types=[
            pltpu.VMEM((NCH // 2, CH), jnp.int32),
            pltpu.VMEM((NCH // 2, CH), jnp.int32),
            pltpu.VMEM((CH, F), jnp.float32),
            pltpu.VMEM((CH, F), jnp.float32),
            pltpu.VMEM_SHARED((NP, F), jnp.float32),
            pltpu.SemaphoreType.DMA,
            pltpu.SemaphoreType.DMA,
        ],
    )
    def agg(src_hbm, dst_hbm, h_hbm, zz_hbm, out_hbm,
            idx_s, idx_d, rows0, rows1, acc, sem0, sem1):
        cid = lax.axis_index("c")
        sid = lax.axis_index("s")
        tid = cid * 16 + sid
        pltpu.sync_copy(zz_hbm, rows0)
        for k in range(RPT // CH):
            pltpu.sync_copy(rows0, acc.at[pl.ds(sid * RPT + k * CH, CH)])
        plsc.subcore_barrier()

        rows = (rows0, rows1)
        sems = (sem0, sem1)
        HC = NCH // 2

        def half(h, carry):
            pltpu.sync_copy(src_hbm.at[pl.ds(tid * NCH + h * HC, HC)], idx_s)
            pltpu.sync_copy(dst_hbm.at[pl.ds(tid * NCH + h * HC, HC)], idx_d)

            def body(i, c):
                j0 = i * 2
                ds = [pltpu.async_copy(h_hbm.at[idx_s.at[j0 + b]],
                                       rows[b], sems[b])
                      for b in range(2)]
                for b in range(2):
                    ds[b].wait()
                    pltpu.sync_copy(rows[b], acc.at[idx_d.at[j0 + b]], add=True)
                return c

            lax.fori_loop(0, HC // 2, body, 0)
            return carry

        lax.fori_loop(0, 2, half, 0)
        plsc.subcore_barrier()
        for k in range(RPT // CH):
            r0 = sid * RPT + k * CH
            pltpu.sync_copy(acc.at[pl.ds(r0, CH)], rows0)
            pltpu.sync_copy(rows0, out_hbm.at[pl.ds(cid * NP + r0, CH)])

    return agg


_agg = _make_agg(128)


@functools.partial(
    pl.kernel,
    mesh=_mesh,
    out_type=(
        jax.ShapeDtypeStruct((2 * NP,), jnp.float32),
        jax.ShapeDtypeStruct((2 * NP,), jnp.float32),
    ),
    scratch_types=[
        pltpu.VMEM((NCH, CH), jnp.int32),
        pltpu.VMEM((NCH, CH), jnp.int32),
        pltpu.VMEM((CH,), jnp.float32),
        pltpu.VMEM((RPT,), jnp.float32),
        pltpu.VMEM_SHARED((NP,), jnp.float32),
        pltpu.VMEM_SHARED((NP,), jnp.float32),
    ],
)
def _deg(src_hbm, dst_hbm, ones_hbm, zz_hbm, out_o, out_i,
         idx_s, idx_d, ones_v, tmp, acc_o, acc_i):
    cid = lax.axis_index("c")
    sid = lax.axis_index("s")
    tid = cid * 16 + sid
    pltpu.sync_copy(ones_hbm, ones_v)
    pltpu.sync_copy(zz_hbm, tmp)
    pltpu.sync_copy(tmp, acc_o.at[pl.ds(sid * RPT, RPT)])
    pltpu.sync_copy(tmp, acc_i.at[pl.ds(sid * RPT, RPT)])
    pltpu.sync_copy(src_hbm.at[pl.ds(tid * NCH, NCH)], idx_s)
    pltpu.sync_copy(dst_hbm.at[pl.ds(tid * NCH, NCH)], idx_d)
    plsc.subcore_barrier()

    def body(j, carry):
        pltpu.sync_copy(ones_v, acc_o.at[idx_s.at[j]], add=True)
        pltpu.sync_copy(ones_v, acc_i.at[idx_d.at[j]], add=True)
        return carry

    lax.fori_loop(0, NCH, body, 0)
    plsc.subcore_barrier()
    pltpu.sync_copy(acc_o.at[pl.ds(sid * RPT, RPT)], tmp)
    pltpu.sync_copy(tmp, out_o.at[pl.ds(cid * NP + sid * RPT, RPT)])
    pltpu.sync_copy(acc_i.at[pl.ds(sid * RPT, RPT)], tmp)
    pltpu.sync_copy(tmp, out_i.at[pl.ds(cid * NP + sid * RPT, RPT)])


def _elu(v):
    return jnp.where(v > 0, v, jnp.exp(jnp.minimum(v, 0.0)) - 1.0)


def _norm(d_ref):
    d = d_ref[0] + d_ref[1]
    return lax.rsqrt(jnp.maximum(d, 1.0))


def _tc1_body(x_ref, w_ref, do_ref, o_ref):
    ns = _norm(do_ref)
    o_ref[...] = jnp.dot(x_ref[...] * ns, w_ref[...],
                         preferred_element_type=jnp.float32)


def _tc2_body(a_ref, b_ref, di_ref, do_ref, o_ref):
    nd = _norm(di_ref)
    ns = _norm(do_ref)
    a = a_ref[0] + a_ref[1]
    o_ref[...] = _elu(a * nd + b_ref[...]) * ns


def _tc3_body(a_ref, w2_ref, b2_ref, w3_ref, di_ref, do_ref, o_ref):
    nd = _norm(di_ref)
    ns = _norm(do_ref)
    a = a_ref[0] + a_ref[1]
    h = _elu(jnp.dot(a * nd, w2_ref[...], preferred_element_type=jnp.float32)
             + b2_ref[...])
    o_ref[...] = jnp.dot(h * ns, w3_ref[...], preferred_element_type=jnp.float32)


def _tc4_body(a_ref, b_ref, di_ref, o_ref):
    nd = _norm(di_ref)
    a = a_ref[0] + a_ref[1]
    v = a * nd + b_ref[...]
    o_ref[...] = 1.0 / (1.0 + jnp.exp(-v))


def _deg_spec():
    return pl.BlockSpec((2, BR, 1), lambda i: (0, i, 0))


def _resh2(a):
    return a.reshape(2, NP, a.shape[-1])


def kernel(edge_index, x, W1, b1, W2, b2, W3, b3):
    src = edge_index[0].astype(jnp.int32)
    dst = edge_index[1].astype(jnp.int32)
    pad_e = PE - E
    pad_row = N + jnp.arange(pad_e, dtype=jnp.int32) % (NP - N)
    srcp = jnp.concatenate([src, pad_row]).reshape(TCH, CH)
    dstp = jnp.concatenate([dst, pad_row]).reshape(TCH, CH)
    xp = jnp.pad(x, ((0, NP - N), (0, 0)))
    W1p = jnp.pad(W1, ((0, 0), (0, 28)))
    b1p = jnp.pad(b1, (0, 28)).reshape(1, 128)
    W2p = jnp.pad(W2, ((0, 28), (0, 0)))
    b2r = b2.reshape(1, 200)
    b3r = b3.reshape(1, 128)
    ones_c = jnp.ones((CH,), jnp.float32)
    zz_r = jnp.zeros((RPT,), jnp.float32)
    zz128 = jnp.zeros((CH, 128), jnp.float32)

    deg_o, deg_i = _deg(srcp, dstp, ones_c, zz_r)
    deg_o = deg_o.reshape(2, NP, 1)
    deg_i = deg_i.reshape(2, NP, 1)

    t1 = pl.pallas_call(
        _tc1_body,
        grid=(NP // BR,),
        in_specs=[
            pl.BlockSpec((BR, 128), lambda i: (i, 0)),
            pl.BlockSpec((128, 128), lambda i: (0, 0)),
            _deg_spec(),
        ],
        out_specs=pl.BlockSpec((BR, 128), lambda i: (i, 0)),
        out_shape=jax.ShapeDtypeStruct((NP, 128), jnp.float32),
    )(xp, W1p, deg_o)

    a1 = _resh2(_agg(srcp, dstp, t1, zz128))

    u2 = pl.pallas_call(
        _tc2_body,
        grid=(NP // BR,),
        in_specs=[
            pl.BlockSpec((2, BR, 128), lambda i: (0, i, 0)),
            pl.BlockSpec((1, 128), lambda i: (0, 0)),
            _deg_spec(),
            _deg_spec(),
        ],
        out_specs=pl.BlockSpec((BR, 128), lambda i: (i, 0)),
        out_shape=jax.ShapeDtypeStruct((NP, 128), jnp.float32),
    )(a1, b1p, deg_i, deg_o)

    a2 = _resh2(_agg(srcp, dstp, u2, zz128))

    t3 = pl.pallas_call(
        _tc3_body,
        grid=(NP // BR,),
        in_specs=[
            pl.BlockSpec((2, BR, 128), lambda i: (0, i, 0)),
            pl.BlockSpec((128, 200), lambda i: (0, 0)),
            pl.BlockSpec((1, 200), lambda i: (0, 0)),
            pl.BlockSpec((200, 128), lambda i: (0, 0)),
            _deg_spec(),
            _deg_spec(),
        ],
        out_specs=pl.BlockSpec((BR, 128), lambda i: (i, 0)),
        out_shape=jax.ShapeDtypeStruct((NP, 128), jnp.float32),
    )(a2, W2p, b2r, W3, deg_i, deg_o)

    a3 = _resh2(_agg(srcp, dstp, t3, zz128))

    out = pl.pallas_call(
        _tc4_body,
        grid=(NP // BR,),
        in_specs=[
            pl.BlockSpec((2, BR, 128), lambda i: (0, i, 0)),
            pl.BlockSpec((1, 128), lambda i: (0, 0)),
            _deg_spec(),
        ],
        out_specs=pl.BlockSpec((BR, 128), lambda i: (i, 0)),
        out_shape=jax.ShapeDtypeStruct((NP, 128), jnp.float32),
    )(a3, b3r, deg_i)

    return out[:N]


# speedup vs baseline: 11.4131x; 1.2458x over previous
import functools

import jax
import jax.numpy as jnp
from jax import lax
from jax.experimental import pallas as pl
from jax.experimental.pallas import tpu as pltpu
from jax.experimental.pallas import tpu_sc as plsc

N = 10000
NP = 10240
E = 320000
CH = 128
NCH = 80
EPT = NCH * CH
NT = 32
PE = NT * EPT
TCH = PE // CH
RPT = NP // 16
BR = 1024

_mesh = plsc.VectorSubcoreMesh(core_axis_name="c", subcore_axis_name="s")


def _make_agg(F):
    @functools.partial(
        pl.kernel,
        mesh=_mesh,
        out_type=jax.ShapeDtypeStruct((2 * NP, F), jnp.float32),
        scratch_types=[
            pltpu.VMEM((NCH // 2, CH), jnp.int32),
            pltpu.VMEM((NCH // 2, CH), jnp.int32),
            pltpu.VMEM((CH, F), jnp.float32),
            pltpu.VMEM((CH, F), jnp.float32),
            pltpu.VMEM_SHARED((NP, F), jnp.float32),
            pltpu.SemaphoreType.DMA,
            pltpu.SemaphoreType.DMA,
        ],
    )
    def agg(src_hbm, dst_hbm, h_hbm, zz_hbm, out_hbm,
            idx_s, idx_d, rows0, rows1, acc, sem0, sem1):
        cid = lax.axis_index("c")
        sid = lax.axis_index("s")
        tid = cid * 16 + sid
        pltpu.sync_copy(zz_hbm, rows0)
        for k in range(RPT // CH):
            pltpu.sync_copy(rows0, acc.at[pl.ds(sid * RPT + k * CH, CH)])
        plsc.subcore_barrier()

        rows = (rows0, rows1)
        sems = (sem0, sem1)
        HC = NCH // 2

        def gwait(b):
            pltpu.make_async_copy(h_hbm.at[pl.ds(0, CH)], rows[b], sems[b]).wait()

        def half(h, carry):
            pltpu.sync_copy(src_hbm.at[pl.ds(tid * NCH + h * HC, HC)], idx_s)
            pltpu.sync_copy(dst_hbm.at[pl.ds(tid * NCH + h * HC, HC)], idx_d)
            for b in range(2):
                pltpu.async_copy(h_hbm.at[idx_s.at[b]], rows[b], sems[b])

            def body(i, c):
                j0 = i * 2
                for b in range(2):
                    gwait(b)
                    pltpu.sync_copy(rows[b], acc.at[idx_d.at[j0 + b]], add=True)
                    nxt = jnp.minimum(j0 + 2 + b, HC - 1)
                    pltpu.async_copy(h_hbm.at[idx_s.at[nxt]], rows[b], sems[b])
                return c

            lax.fori_loop(0, HC // 2, body, 0)
            for b in range(2):
                gwait(b)
            return carry

        lax.fori_loop(0, 2, half, 0)
        plsc.subcore_barrier()
        for k in range(RPT // CH):
            r0 = sid * RPT + k * CH
            pltpu.sync_copy(acc.at[pl.ds(r0, CH)], rows0)
            pltpu.sync_copy(rows0, out_hbm.at[pl.ds(cid * NP + r0, CH)])

    return agg


_agg = _make_agg(128)


@functools.partial(
    pl.kernel,
    mesh=_mesh,
    out_type=(
        jax.ShapeDtypeStruct((2 * NP,), jnp.float32),
        jax.ShapeDtypeStruct((2 * NP,), jnp.float32),
    ),
    scratch_types=[
        pltpu.VMEM((NCH, CH), jnp.int32),
        pltpu.VMEM((NCH, CH), jnp.int32),
        pltpu.VMEM((CH,), jnp.float32),
        pltpu.VMEM((RPT,), jnp.float32),
        pltpu.VMEM_SHARED((NP,), jnp.float32),
        pltpu.VMEM_SHARED((NP,), jnp.float32),
    ],
)
def _deg(src_hbm, dst_hbm, ones_hbm, zz_hbm, out_o, out_i,
         idx_s, idx_d, ones_v, tmp, acc_o, acc_i):
    cid = lax.axis_index("c")
    sid = lax.axis_index("s")
    tid = cid * 16 + sid
    pltpu.sync_copy(ones_hbm, ones_v)
    pltpu.sync_copy(zz_hbm, tmp)
    pltpu.sync_copy(tmp, acc_o.at[pl.ds(sid * RPT, RPT)])
    pltpu.sync_copy(tmp, acc_i.at[pl.ds(sid * RPT, RPT)])
    pltpu.sync_copy(src_hbm.at[pl.ds(tid * NCH, NCH)], idx_s)
    pltpu.sync_copy(dst_hbm.at[pl.ds(tid * NCH, NCH)], idx_d)
    plsc.subcore_barrier()

    def body(j, carry):
        pltpu.sync_copy(ones_v, acc_o.at[idx_s.at[j]], add=True)
        pltpu.sync_copy(ones_v, acc_i.at[idx_d.at[j]], add=True)
        return carry

    lax.fori_loop(0, NCH, body, 0)
    plsc.subcore_barrier()
    pltpu.sync_copy(acc_o.at[pl.ds(sid * RPT, RPT)], tmp)
    pltpu.sync_copy(tmp, out_o.at[pl.ds(cid * NP + sid * RPT, RPT)])
    pltpu.sync_copy(acc_i.at[pl.ds(sid * RPT, RPT)], tmp)
    pltpu.sync_copy(tmp, out_i.at[pl.ds(cid * NP + sid * RPT, RPT)])


def _elu(v):
    return jnp.where(v > 0, v, jnp.exp(jnp.minimum(v, 0.0)) - 1.0)


def _norm(d_ref):
    d = d_ref[0] + d_ref[1]
    return lax.rsqrt(jnp.maximum(d, 1.0))


def _tc1_body(x_ref, w_ref, do_ref, o_ref):
    ns = _norm(do_ref)
    o_ref[...] = jnp.dot(x_ref[...] * ns, w_ref[...],
                         preferred_element_type=jnp.float32)


def _tc2_body(a_ref, b_ref, di_ref, do_ref, o_ref):
    nd = _norm(di_ref)
    ns = _norm(do_ref)
    a = a_ref[0] + a_ref[1]
    o_ref[...] = _elu(a * nd + b_ref[...]) * ns


def _tc3_body(a_ref, w2_ref, b2_ref, w3_ref, di_ref, do_ref, o_ref):
    nd = _norm(di_ref)
    ns = _norm(do_ref)
    a = a_ref[0] + a_ref[1]
    h = _elu(jnp.dot(a * nd, w2_ref[...], preferred_element_type=jnp.float32)
             + b2_ref[...])
    o_ref[...] = jnp.dot(h * ns, w3_ref[...], preferred_element_type=jnp.float32)


def _tc4_body(a_ref, b_ref, di_ref, o_ref):
    nd = _norm(di_ref)
    a = a_ref[0] + a_ref[1]
    v = a * nd + b_ref[...]
    o_ref[...] = 1.0 / (1.0 + jnp.exp(-v))


def _deg_spec():
    return pl.BlockSpec((2, BR, 1), lambda i: (0, i, 0))


def _resh2(a):
    return a.reshape(2, NP, a.shape[-1])


def kernel(edge_index, x, W1, b1, W2, b2, W3, b3):
    src = edge_index[0].astype(jnp.int32)
    dst = edge_index[1].astype(jnp.int32)
    pad_e = PE - E
    pad_row = N + jnp.arange(pad_e, dtype=jnp.int32) % (NP - N)
    srcp = jnp.concatenate([src, pad_row]).reshape(TCH, CH)
    dstp = jnp.concatenate([dst, pad_row]).reshape(TCH, CH)
    xp = jnp.pad(x, ((0, NP - N), (0, 0)))
    W1p = jnp.pad(W1, ((0, 0), (0, 28)))
    b1p = jnp.pad(b1, (0, 28)).reshape(1, 128)
    W2p = jnp.pad(W2, ((0, 28), (0, 0)))
    b2r = b2.reshape(1, 200)
    b3r = b3.reshape(1, 128)
    ones_c = jnp.ones((CH,), jnp.float32)
    zz_r = jnp.zeros((RPT,), jnp.float32)
    zz128 = jnp.zeros((CH, 128), jnp.float32)

    deg_o, deg_i = _deg(srcp, dstp, ones_c, zz_r)
    deg_o = deg_o.reshape(2, NP, 1)
    deg_i = deg_i.reshape(2, NP, 1)

    t1 = pl.pallas_call(
        _tc1_body,
        grid=(NP // BR,),
        in_specs=[
            pl.BlockSpec((BR, 128), lambda i: (i, 0)),
            pl.BlockSpec((128, 128), lambda i: (0, 0)),
            _deg_spec(),
        ],
        out_specs=pl.BlockSpec((BR, 128), lambda i: (i, 0)),
        out_shape=jax.ShapeDtypeStruct((NP, 128), jnp.float32),
    )(xp, W1p, deg_o)

    a1 = _resh2(_agg(srcp, dstp, t1, zz128))

    u2 = pl.pallas_call(
        _tc2_body,
        grid=(NP // BR,),
        in_specs=[
            pl.BlockSpec((2, BR, 128), lambda i: (0, i, 0)),
            pl.BlockSpec((1, 128), lambda i: (0, 0)),
            _deg_spec(),
            _deg_spec(),
        ],
        out_specs=pl.BlockSpec((BR, 128), lambda i: (i, 0)),
        out_shape=jax.ShapeDtypeStruct((NP, 128), jnp.float32),
    )(a1, b1p, deg_i, deg_o)

    a2 = _resh2(_agg(srcp, dstp, u2, zz128))

    t3 = pl.pallas_call(
        _tc3_body,
        grid=(NP // BR,),
        in_specs=[
            pl.BlockSpec((2, BR, 128), lambda i: (0, i, 0)),
            pl.BlockSpec((128, 200), lambda i: (0, 0)),
            pl.BlockSpec((1, 200), lambda i: (0, 0)),
            pl.BlockSpec((200, 128), lambda i: (0, 0)),
            _deg_spec(),
            _deg_spec(),
        ],
        out_specs=pl.BlockSpec((BR, 128), lambda i: (i, 0)),
        out_shape=jax.ShapeDtypeStruct((NP, 128), jnp.float32),
    )(a2, W2p, b2r, W3, deg_i, deg_o)

    a3 = _resh2(_agg(srcp, dstp, t3, zz128))

    out = pl.pallas_call(
        _tc4_body,
        grid=(NP // BR,),
        in_specs=[
            pl.BlockSpec((2, BR, 128), lambda i: (0, i, 0)),
            pl.BlockSpec((1, 128), lambda i: (0, 0)),
            _deg_spec(),
        ],
        out_specs=pl.BlockSpec((BR, 128), lambda i: (i, 0)),
        out_shape=jax.ShapeDtypeStruct((NP, 128), jnp.float32),
    )(a3, b3r, deg_i)

    return out[:N]
